# baseline (speedup 1.0000x reference)
import jax
import jax.numpy as jnp
from jax import lax
from jax.experimental import pallas as pl
from jax.experimental.pallas import tpu as pltpu

_B, _S, _F = 16, 2048, 128
_RB = 256
_NSTEP = _S // _RB
_BB = _B // _NSTEP
_NS = 8
_NL = _S // _NS
_BIG = float(jnp.finfo(jnp.float32).max)


def _min_tree(vals):
    while len(vals) > 1:
        nxt = [jnp.minimum(vals[k], vals[k + 1])
               for k in range(0, len(vals) - 1, 2)]
        if len(vals) % 2:
            nxt.append(vals[-1])
        vals = nxt
    return vals[0]


def _dotf(a, b, dims):
    return lax.dot_general(a, b, (dims, ((), ())),
                           precision=lax.Precision.HIGHEST,
                           preferred_element_type=jnp.float32)


def _sshift(v, k):
    r = pltpu.roll(v, k, axis=0)
    mask = lax.broadcasted_iota(jnp.int32, v.shape, 0) < k
    return jnp.where(mask, _BIG, r)


def _lshift(v, m):
    r = pltpu.roll(v, m, axis=1)
    mask = lax.broadcasted_iota(jnp.int32, v.shape, 1) < m
    return jnp.where(mask, _BIG, r)


def _cummin_cm(v):
    for k in (1, 2, 4):
        v = jnp.minimum(v, _sshift(v, k))
    t = v[_NS - 1:_NS, :]
    te = _min_tree([_lshift(t, m) for m in range(1, 17)])
    te = _min_tree([te] + [_lshift(te, 16 * m) for m in range(1, 16)])
    return jnp.minimum(v, te)


def _row_update(prev, drow, c):
    m1 = jnp.minimum(prev, _sshift(prev, 1))
    r7 = _lshift(prev[_NS - 1:_NS, :], 1)
    sub0 = lax.broadcasted_iota(jnp.int32, (_NS, _NL), 0) == 0
    m = jnp.where(sub0, jnp.minimum(m1, r7), m1)
    b = drow + m
    return c + _cummin_cm(b - c)


def _fused_kernel(pred_ref, target_ref, x_ref, y_ref,
                  perm_ref, lp_ref, uex_ref, out_ref,
                  g_scr, dprev_scr, acc_ref):
    i = pl.program_id(0)

    part = jnp.sum(jnp.abs(pred_ref[...] - target_ref[...]))

    @pl.when(i == 0)
    def _():
        acc_ref[0] = 0.0

    acc_ref[0] = acc_ref[0] + part

    xb = x_ref[...]
    y = y_ref[...]
    xsq = jnp.sum(xb * xb, axis=1, keepdims=True)
    ysq = jnp.sum(y * y, axis=1, keepdims=True)
    lhs = jnp.concatenate(
        [-2.0 * xb, xsq, jnp.ones((_RB, 1), jnp.float32)], axis=1)
    rhs = jnp.concatenate(
        [y, jnp.ones((_S, 1), jnp.float32), ysq], axis=1)
    sq = lax.dot_general(lhs, rhs, (((1,), (1,)), ((), ())),
                         preferred_element_type=jnp.float32)
    d = jnp.sqrt(jnp.maximum(sq, 1e-12))
    for s in range(_NS):
        g_scr[s] = d[:, s * _NL:(s + 1) * _NL]

    perm = perm_ref[...]
    lp = lp_ref[...]
    uex = uex_ref[...]

    def load_group(base):
        s_cat = g_scr[:, pl.ds(base, 8), :].reshape(_NS * 8, _NL)
        v_cat = _dotf(perm, s_cat, ((1,), (0,)))
        incol = _dotf(lp, s_cat, ((1,), (0,)))
        tall = jnp.concatenate(
            [incol[8 * r8 + 7:8 * r8 + 8, :] for r8 in range(8)],
            axis=0)
        te_all = _dotf(tall, uex, ((1,), (0,)))
        rows = [v_cat[8 * r8:8 * r8 + 8, :] for r8 in range(8)]
        cs = [incol[8 * r8:8 * r8 + 8, :] + te_all[r8:r8 + 1, :]
              for r8 in range(8)]
        return rows, cs

    @pl.when(i == 0)
    def _():
        rows, cs = load_group(0)
        row = cs[0]
        for r8 in range(1, 8):
            row = _row_update(row, rows[r8], cs[r8])
        dprev_scr[...] = row

    start = jnp.where(i == 0, 1, 0)

    def outer(rt, carry):
        base = pl.multiple_of(rt * 8, 8)
        rows, cs = load_group(base)
        for r8 in range(8):
            carry = _row_update(carry, rows[r8], cs[r8])
        return carry

    final = lax.fori_loop(start, _RB // 8, outer, dprev_scr[...])
    dprev_scr[...] = final

    @pl.when(i == _NSTEP - 1)
    def _():
        mae = acc_ref[0] / float(_B * _S * _F)
        dtw = final[_NS - 1, _NL - 1] / float(_S * _F)
        out_ref[...] = (0.5 * mae + 0.5 * dtw) * jnp.ones((1, 1), jnp.float32)


def kernel(pred, target):
    x = pred[0]
    pj = (jnp.arange(_S, dtype=jnp.int32) % _NL) * _NS \
        + jnp.arange(_S, dtype=jnp.int32) // _NL
    y = target[0][pj]

    k = jnp.arange(64)
    ksrc = (k % 8) * 8 + k // 8
    perm = (ksrc[:, None] == jnp.arange(64)[None, :]).astype(jnp.float32)
    lbd = ((k[:, None] // 8 == k[None, :] // 8)
           & (k[None, :] % 8 <= k[:, None] % 8)).astype(jnp.float32)
    lp = lbd @ perm
    cl = jnp.arange(_NL)
    uex = (cl[:, None] < cl[None, :]).astype(jnp.float32)

    out = pl.pallas_call(
        _fused_kernel,
        grid=(_NSTEP,),
        in_specs=[
            pl.BlockSpec((_BB, _S, _F), lambda i: (i, 0, 0)),
            pl.BlockSpec((_BB, _S, _F), lambda i: (i, 0, 0)),
            pl.BlockSpec((_RB, _F), lambda i: (i, 0)),
            pl.BlockSpec((_S, _F), lambda i: (0, 0)),
            pl.BlockSpec((64, 64), lambda i: (0, 0)),
            pl.BlockSpec((64, 64), lambda i: (0, 0)),
            pl.BlockSpec((_NL, _NL), lambda i: (0, 0)),
        ],
        out_specs=pl.BlockSpec((1, 1), lambda i: (0, 0)),
        out_shape=jax.ShapeDtypeStruct((1, 1), jnp.float32),
        scratch_shapes=[
            pltpu.VMEM((_NS, _RB, _NL), jnp.float32),
            pltpu.VMEM((_NS, _NL), jnp.float32),
            pltpu.SMEM((1,), jnp.float32),
        ],
        compiler_params=pltpu.CompilerParams(
            dimension_semantics=("arbitrary",),
        ),
    )(pred, target, x, y, perm, lp, uex)
    return out[0, 0]


# speedup vs baseline: 1.1568x; 1.1568x over previous
import jax
import jax.numpy as jnp
from jax import lax
from jax.experimental import pallas as pl
from jax.experimental.pallas import tpu as pltpu

_B, _S, _F = 16, 2048, 128
_RB = 256
_NSTEP = _S // _RB
_BB = _B // _NSTEP
_NS = 8
_NL = _S // _NS
_BIG = float(jnp.finfo(jnp.float32).max)


def _min_tree(vals):
    while len(vals) > 1:
        nxt = [jnp.minimum(vals[k], vals[k + 1])
               for k in range(0, len(vals) - 1, 2)]
        if len(vals) % 2:
            nxt.append(vals[-1])
        vals = nxt
    return vals[0]


def _dotf(a, b, dims):
    return lax.dot_general(a, b, (dims, ((), ())),
                           precision=lax.Precision.HIGHEST,
                           preferred_element_type=jnp.float32)


def _sshift(v, k):
    r = pltpu.roll(v, k, axis=0)
    mask = lax.broadcasted_iota(jnp.int32, v.shape, 0) < k
    return jnp.where(mask, _BIG, r)


def _lshift(v, m):
    r = pltpu.roll(v, m, axis=1)
    mask = lax.broadcasted_iota(jnp.int32, v.shape, 1) < m
    return jnp.where(mask, _BIG, r)


def _sub0():
    return lax.broadcasted_iota(jnp.int32, (_NS, _NL), 0) == 0


def _fs1(v):
    r = pltpu.roll(v, 1, axis=0)
    r7s = _lshift(v[_NS - 1:_NS, :], 1)
    return jnp.where(_sub0(), r7s, r)


def _row_step(carry, drow, c):
    te, te_s1, e1, a, cprev, _w = carry
    bmix = jnp.where(_sub0(), te_s1, te)
    q = jnp.minimum(cprev + te, a + bmix)
    m = jnp.minimum(e1, q)
    b = drow + m
    w = b - c
    for k in (1, 2, 4):
        w = jnp.minimum(w, _sshift(w, k))
    t = w[_NS - 1:_NS, :]
    te1 = _min_tree([_lshift(t, mm) for mm in range(1, 17)])
    te_n = _min_tree([te1] + [_lshift(te1, 16 * mm) for mm in range(1, 16)])
    tes1_n = _min_tree([_lshift(te1, 16 * mm + 1) for mm in range(16)])
    p = c + w
    e1_n = jnp.minimum(p, _fs1(p))
    a_n = _fs1(c)
    return (te_n, tes1_n, e1_n, a_n, c, w)


def _init_carry(c0):
    te0 = jnp.full((1, _NL), _BIG, jnp.float32)
    return (te0, te0, jnp.minimum(c0, _fs1(c0)), _fs1(c0), c0,
            jnp.zeros((_NS, _NL), jnp.float32))


def _fused_kernel(pred_ref, target_ref, x_ref, y_ref,
                  perm_ref, lp_ref, uex_ref, out_ref,
                  g_scr, st_scr, acc_ref):
    i = pl.program_id(0)

    part = jnp.sum(jnp.abs(pred_ref[...] - target_ref[...]))

    @pl.when(i == 0)
    def _():
        acc_ref[0] = 0.0

    acc_ref[0] = acc_ref[0] + part

    xb = x_ref[...]
    y = y_ref[...]
    xsq = jnp.sum(xb * xb, axis=1, keepdims=True)
    ysq = jnp.sum(y * y, axis=1, keepdims=True)
    lhs = jnp.concatenate(
        [-2.0 * xb, xsq, jnp.ones((_RB, 1), jnp.float32)], axis=1)
    rhs = jnp.concatenate(
        [y, jnp.ones((_S, 1), jnp.float32), ysq], axis=1)
    sq = lax.dot_general(lhs, rhs, (((1,), (1,)), ((), ())),
                         preferred_element_type=jnp.float32)
    d = jnp.sqrt(jnp.maximum(sq, 1e-12))
    for s in range(_NS):
        g_scr[s] = d[:, s * _NL:(s + 1) * _NL]

    perm = perm_ref[...]
    lp = lp_ref[...]
    uex = uex_ref[...]

    def load_group(base):
        s_cat = g_scr[:, pl.ds(base, 8), :].reshape(_NS * 8, _NL)
        v_cat = _dotf(perm, s_cat, ((1,), (0,)))
        incol = _dotf(lp, s_cat, ((1,), (0,)))
        tall = jnp.concatenate(
            [incol[8 * r8 + 7:8 * r8 + 8, :] for r8 in range(8)],
            axis=0)
        te_all = _dotf(tall, uex, ((1,), (0,)))
        rows = [v_cat[8 * r8:8 * r8 + 8, :] for r8 in range(8)]
        cs = [incol[8 * r8:8 * r8 + 8, :] + te_all[r8:r8 + 1, :]
              for r8 in range(8)]
        return rows, cs

    def _store_carry(carry):
        te, tes1, e1, a, c, w = carry
        st_scr[0, 0:1, :] = te
        st_scr[0, 1:2, :] = tes1
        st_scr[1] = e1
        st_scr[2] = a
        st_scr[3] = c
        st_scr[4] = w

    @pl.when(i == 0)
    def _():
        rows, cs = load_group(0)
        carry = _init_carry(cs[0])
        for r8 in range(1, 8):
            carry = _row_step(carry, rows[r8], cs[r8])
        _store_carry(carry)

    start = jnp.where(i == 0, 1, 0)
    carry_in = (st_scr[0, 0:1, :], st_scr[0, 1:2, :],
                st_scr[1], st_scr[2], st_scr[3], st_scr[4])

    def outer(rt, carry):
        base = pl.multiple_of(rt * 8, 8)
        rows, cs = load_group(base)
        for r8 in range(8):
            carry = _row_step(carry, rows[r8], cs[r8])
        return carry

    final = lax.fori_loop(start, _RB // 8, outer, carry_in)
    _store_carry(final)

    @pl.when(i == _NSTEP - 1)
    def _():
        te_f, _, _, _, c_f, w_f = final
        mae = acc_ref[0] / float(_B * _S * _F)
        dtw_last = c_f[_NS - 1, _NL - 1] + jnp.minimum(
            w_f[_NS - 1, _NL - 1], te_f[0, _NL - 1])
        dtw = dtw_last / float(_S * _F)
        out_ref[...] = (0.5 * mae + 0.5 * dtw) * jnp.ones((1, 1), jnp.float32)


def kernel(pred, target):
    x = pred[0]
    pj = (jnp.arange(_S, dtype=jnp.int32) % _NL) * _NS \
        + jnp.arange(_S, dtype=jnp.int32) // _NL
    y = target[0][pj]

    k = jnp.arange(64)
    ksrc = (k % 8) * 8 + k // 8
    perm = (ksrc[:, None] == jnp.arange(64)[None, :]).astype(jnp.float32)
    lbd = ((k[:, None] // 8 == k[None, :] // 8)
           & (k[None, :] % 8 <= k[:, None] % 8)).astype(jnp.float32)
    lp = lbd @ perm
    cl = jnp.arange(_NL)
    uex = (cl[:, None] < cl[None, :]).astype(jnp.float32)

    out = pl.pallas_call(
        _fused_kernel,
        grid=(_NSTEP,),
        in_specs=[
            pl.BlockSpec((_BB, _S, _F), lambda i: (i, 0, 0)),
            pl.BlockSpec((_BB, _S, _F), lambda i: (i, 0, 0)),
            pl.BlockSpec((_RB, _F), lambda i: (i, 0)),
            pl.BlockSpec((_S, _F), lambda i: (0, 0)),
            pl.BlockSpec((64, 64), lambda i: (0, 0)),
            pl.BlockSpec((64, 64), lambda i: (0, 0)),
            pl.BlockSpec((_NL, _NL), lambda i: (0, 0)),
        ],
        out_specs=pl.BlockSpec((1, 1), lambda i: (0, 0)),
        out_shape=jax.ShapeDtypeStruct((1, 1), jnp.float32),
        scratch_shapes=[
            pltpu.VMEM((_NS, _RB, _NL), jnp.float32),
            pltpu.VMEM((5, _NS, _NL), jnp.float32),
            pltpu.SMEM((1,), jnp.float32),
        ],
        compiler_params=pltpu.CompilerParams(
            dimension_semantics=("arbitrary",),
        ),
    )(pred, target, x, y, perm, lp, uex)
    return out[0, 0]
